# baseline (speedup 1.0000x reference)
import functools

import jax
import jax.numpy as jnp
from jax import lax
from jax.experimental import pallas as pl
from jax.experimental.pallas import tpu as pltpu
from jax.experimental.pallas import tpu_sc as plsc

B = 64
V = 1_000_000
MAIN_COLS = 999_936
TAIL_COLS = V - MAIN_COLS
NSTRIPES = 4
STRIPE = MAIN_COLS // NSTRIPES
CK = 2688
NCH = STRIPE // CK
UNROLL = 8
NEG_LN2 = -0.6931471805599453
I32_MAX = 2147483647

_LOG2_COEF = (
    0.014598474837839603, -0.07592024654150009, 0.18865151703357697,
    -0.32148241996765137, 0.4717213213443756, -0.7202025651931763,
    1.4426336288452148, 8.121997439047846e-07,
)


def _score16(lv, nv, c2v):
    n1 = nv + jnp.float32(1e-10)
    bits = plsc.bitcast(n1, jnp.int32)
    e_f = jnp.right_shift(bits, 23).astype(jnp.float32) - jnp.float32(127.0)
    u = jnp.bitwise_and(bits, 0x7FFFFF).astype(jnp.float32) * jnp.float32(2.0 ** -23)
    c7, c6, c5, c4, c3, c2, c1, c0 = (jnp.float32(x) for x in _LOG2_COEF)
    u2 = u * u
    u4 = u2 * u2
    pa = c7 * u + c6
    pb = c5 * u + c4
    pc = c3 * u + c2
    pd = c1 * u + c0
    pab = pa * u2 + pb
    pcd = pc * u2 + pd
    p = pab * u4 + pcd
    s = c2v * p + lv
    s = c2v * e_f + s
    return s


def _update(m, bi, idx, s):
    take = s > m
    bi = jnp.where(take, idx, bi)
    m = jnp.maximum(m, s)
    return m, bi


def _make_sampler():
    mesh = plsc.VectorSubcoreMesh(core_axis_name="c", subcore_axis_name="s")

    @functools.partial(
        pl.kernel,
        out_type=jax.ShapeDtypeStruct((B,), jnp.int32),
        mesh=mesh,
        scratch_types=[
            pltpu.VMEM((8, CK), jnp.float32),
            pltpu.VMEM((8, CK), jnp.float32),
            pltpu.VMEM((8, CK), jnp.float32),
            pltpu.VMEM((8, CK), jnp.float32),
            pltpu.VMEM((512,), jnp.float32),
            pltpu.VMEM((512,), jnp.float32),
            pltpu.VMEM((128,), jnp.float32),
            pltpu.VMEM((16,), jnp.float32),
            pltpu.VMEM((16,), jnp.int32),
            pltpu.VMEM((64,), jnp.float32),
            pltpu.VMEM((64,), jnp.int32),
            pltpu.VMEM_SHARED((256,), jnp.float32),
            pltpu.VMEM_SHARED((256,), jnp.int32),
            pltpu.SemaphoreType.DMA,
            pltpu.SemaphoreType.DMA,
            pltpu.SemaphoreType.DMA,
        ],
        compiler_params=pltpu.CompilerParams(needs_layout_passes=False),
    )
    def sampler(logits_hbm, temps_hbm, ltail_hbm, ntail_hbm, noise_hbm, out_hbm,
                lbuf0, lbuf1, nbuf0, nbuf1, ltail, ntail, tbuf,
                pmv, piv, mbuf, ibuf, spm_m, spm_i, sem0, sem1, sem2):
        cid = lax.axis_index("c")
        sid = lax.axis_index("s")
        grp = cid * 4 + sid // 4
        stripe = sid % 4
        row0 = grp * 8
        sbase = stripe * STRIPE

        lbufs = (lbuf0, lbuf1)
        nbufs = (nbuf0, nbuf1)
        sems = (sem0, sem1)
        lane = lax.iota(jnp.int32, 16)

        for r in range(8):
            pltpu.make_async_copy(
                temps_hbm.at[pl.ds((row0 + r) * 16, 16)],
                tbuf.at[pl.ds(r * 16, 16)], sem2).start()
        for r in range(8):
            pltpu.make_async_copy(
                temps_hbm.at[pl.ds(0, 16)],
                tbuf.at[pl.ds(r * 16, 16)], sem2).wait()
        c2vs = [tbuf[pl.ds(r * 16, 16)] * jnp.float32(NEG_LN2) for r in range(8)]

        pltpu.make_async_copy(
            ltail_hbm.at[pl.ds(grp * 512, 512)], ltail, sem2).start()
        pltpu.make_async_copy(
            ntail_hbm.at[pl.ds(grp * 512, 512)], ntail, sem2).start()
        pltpu.make_async_copy(
            ltail_hbm.at[pl.ds(0, 512)], ltail, sem2).wait()
        pltpu.make_async_copy(
            ntail_hbm.at[pl.ds(0, 512)], ntail, sem2).wait()

        def start_chunk(c, b):
            col = sbase + c * CK
            pltpu.make_async_copy(
                logits_hbm.at[pl.ds(row0, 8), pl.ds(col, CK)], lbufs[b], sems[b]).start()
            pltpu.make_async_copy(
                noise_hbm.at[pl.ds(row0, 8), pl.ds(col, CK)], nbufs[b], sems[b]).start()

        def wait_chunk(b):
            pltpu.make_async_copy(
                logits_hbm.at[pl.ds(0, 8), pl.ds(0, CK)], lbufs[b], sems[b]).wait()
            pltpu.make_async_copy(
                noise_hbm.at[pl.ds(0, 8), pl.ds(0, CK)], nbufs[b], sems[b]).wait()

        def scan_chunk(c, b, ms, cis):
            out_ms = []
            out_cis = []
            for r in range(8):
                @plsc.parallel_loop(0, CK, step=16 * UNROLL, unroll=1,
                                    carry=jnp.full((16,), -3.0e38, jnp.float32))
                def vbody(off, cm, _r=r, _b=b):
                    leaves = []
                    for k in range(UNROLL):
                        lv = lbufs[_b][_r, pl.ds(off + 16 * k, 16)]
                        nv = nbufs[_b][_r, pl.ds(off + 16 * k, 16)]
                        leaves.append(_score16(lv, nv, c2vs[_r]))
                    while len(leaves) > 1:
                        nxt = [jnp.maximum(leaves[a], leaves[a + 1])
                               for a in range(0, len(leaves) - 1, 2)]
                        if len(leaves) % 2:
                            nxt.append(leaves[-1])
                        leaves = nxt
                    return jnp.maximum(cm, leaves[0])

                cm = vbody
                take = cm > ms[r]
                ci = jnp.where(take, jnp.full((16,), c, jnp.int32), cis[r])
                out_ms.append(jnp.maximum(ms[r], cm))
                out_cis.append(ci)
            return tuple(out_ms), tuple(out_cis)

        ms0 = tuple(jnp.full((16,), -3.0e38, jnp.float32) for _ in range(8))
        cis0 = tuple(jnp.zeros((16,), jnp.int32) for _ in range(8))

        start_chunk(0, 0)
        start_chunk(1, 1)

        def pair_body(cp, carry):
            ms, cis = carry
            c0 = cp * 2
            wait_chunk(0)
            ms, cis = scan_chunk(c0, 0, ms, cis)

            @pl.when(c0 + 2 < NCH)
            def _():
                start_chunk(c0 + 2, 0)

            wait_chunk(1)
            ms, cis = scan_chunk(c0 + 1, 1, ms, cis)

            @pl.when(c0 + 3 < NCH)
            def _():
                start_chunk(c0 + 3, 1)

            return ms, cis

        ms, cis = lax.fori_loop(0, NCH // 2, pair_body, (ms0, cis0))
        wait_chunk(0)
        ms, cis = scan_chunk(NCH - 1, 0, ms, cis)

        tmask = jnp.full((16,), stripe, jnp.int32) == jnp.int32(3)
        neg = jnp.full((16,), -3.0e38, jnp.float32)
        new_ms = []
        new_cis = []
        for r in range(8):
            cm = neg
            for j in range(4):
                lv = ltail[pl.ds(r * 64 + j * 16, 16)]
                nv = ntail[pl.ds(r * 64 + j * 16, 16)]
                s = _score16(lv, nv, c2vs[r])
                cm = jnp.maximum(cm, jnp.where(tmask, s, neg))
            take = cm > ms[r]
            new_cis.append(jnp.where(take, jnp.full((16,), NCH, jnp.int32), cis[r]))
            new_ms.append(jnp.maximum(ms[r], cm))
        ms, cis = tuple(new_ms), tuple(new_cis)

        pm = jnp.full((16,), -3.0e38, jnp.float32)
        pi = jnp.zeros((16,), jnp.int32)
        for r in range(8):
            gmax = jnp.max(ms[r])
            cbest = jnp.min(jnp.where(ms[r] == gmax, cis[r], jnp.int32(I32_MAX)))
            gv = jnp.full((16,), gmax, jnp.float32)
            col0 = sbase + jnp.minimum(cbest, jnp.int32(NCH - 1)) * CK
            pltpu.make_async_copy(
                logits_hbm.at[pl.ds(row0, 8), pl.ds(col0, CK)], lbufs[0], sems[0]).start()
            pltpu.make_async_copy(
                noise_hbm.at[pl.ds(row0, 8), pl.ds(col0, CK)], nbufs[0], sems[0]).start()
            wait_chunk(0)
            idx0 = lane + col0

            @plsc.parallel_loop(0, CK, step=16 * UNROLL, unroll=1,
                                carry=(jnp.full((16,), I32_MAX, jnp.int32), idx0))
            def rbody(off, car, _r=r):
                mn, ib = car
                for k in range(UNROLL):
                    lv = lbufs[0][_r, pl.ds(off + 16 * k, 16)]
                    nv = nbufs[0][_r, pl.ds(off + 16 * k, 16)]
                    s = _score16(lv, nv, c2vs[_r])
                    hit = s == gv
                    mn = jnp.minimum(mn, jnp.where(hit, ib + jnp.int32(16 * k), jnp.full((16,), I32_MAX, jnp.int32)))
                return mn, ib + jnp.int32(16 * UNROLL)

            mn, _ = rbody
            tmn = jnp.full((16,), I32_MAX, jnp.int32)
            for j in range(4):
                lv = ltail[pl.ds(r * 64 + j * 16, 16)]
                nv = ntail[pl.ds(r * 64 + j * 16, 16)]
                s = _score16(lv, nv, c2vs[r])
                hit = s == gv
                tmn = jnp.minimum(tmn, jnp.where(hit, lane + (MAIN_COLS + j * 16), jnp.full((16,), I32_MAX, jnp.int32)))
            mn = jnp.where(jnp.full((16,), cbest, jnp.int32) == NCH, tmn, mn)
            best = jnp.min(mn)
            pm = jnp.where(lane == r, jnp.full((16,), gmax, jnp.float32), pm)
            pi = jnp.where(lane == r, jnp.full((16,), best, jnp.int32), pi)

        pmv[...] = pm
        piv[...] = pi
        pltpu.sync_copy(pmv, spm_m.at[pl.ds(sid * 16, 16)])
        pltpu.sync_copy(piv, spm_i.at[pl.ds(sid * 16, 16)])
        plsc.subcore_barrier()

        @pl.when(stripe == 0)
        def _():
            gbase = (sid // 4) * 64
            pltpu.sync_copy(spm_m.at[pl.ds(gbase, 64)], mbuf)
            pltpu.sync_copy(spm_i.at[pl.ds(gbase, 64)], ibuf)
            m0 = mbuf[pl.ds(0, 16)]
            m1 = mbuf[pl.ds(16, 16)]
            m2 = mbuf[pl.ds(32, 16)]
            m3 = mbuf[pl.ds(48, 16)]
            i0 = ibuf[pl.ds(0, 16)]
            i1 = ibuf[pl.ds(16, 16)]
            i2 = ibuf[pl.ds(32, 16)]
            i3 = ibuf[pl.ds(48, 16)]

            def merge(ma, ia, mb, ib):
                takeb = jnp.logical_or(
                    mb > ma, jnp.logical_and(mb == ma, ib < ia))
                return jnp.where(takeb, mb, ma), jnp.where(takeb, ib, ia)

            ma, ia = merge(m0, i0, m1, i1)
            mb, ib = merge(m2, i2, m3, i3)
            _, fi = merge(ma, ia, mb, ib)
            piv[...] = fi
            pltpu.make_async_copy(
                piv.at[pl.ds(0, 8)], out_hbm.at[pl.ds(row0, 8)], sem2).start()
            pltpu.make_async_copy(
                piv.at[pl.ds(0, 8)], out_hbm.at[pl.ds(row0, 8)], sem2).wait()

    return sampler


_sampler = _make_sampler()


@jax.jit
def kernel(logits, temperatures, exp_noise):
    logits = logits.astype(jnp.float32)
    temps16 = jnp.broadcast_to(
        temperatures.astype(jnp.float32)[:, None], (B, 16)).reshape(B * 16)
    ltail = logits[:, MAIN_COLS:].reshape(B * TAIL_COLS)
    ntail = exp_noise[:, MAIN_COLS:].reshape(B * TAIL_COLS)
    return _sampler(logits, temps16, ltail, ntail, exp_noise)


# speedup vs baseline: 1.1465x; 1.1465x over previous
import functools

import jax
import jax.numpy as jnp
from jax import lax
from jax.experimental import pallas as pl
from jax.experimental.pallas import tpu as pltpu
from jax.experimental.pallas import tpu_sc as plsc

B = 64
V = 1_000_000
MAIN_COLS = 999_936
TAIL_COLS = V - MAIN_COLS
NSTRIPES = 4
STRIPE = MAIN_COLS // NSTRIPES
CK = 2688
NCH = STRIPE // CK
UNROLL = 8
NEG_LN2 = -0.6931471805599453
I32_MAX = 2147483647

_LOG2_COEF = (
    0.014598474837839603, -0.07592024654150009, 0.18865151703357697,
    -0.32148241996765137, 0.4717213213443756, -0.7202025651931763,
    1.4426336288452148, 8.121997439047846e-07,
)


def _score16(lv, nv, c2v):
    n1 = nv + jnp.float32(1e-10)
    bits = plsc.bitcast(n1, jnp.int32)
    e_f = jnp.right_shift(bits, 23).astype(jnp.float32) - jnp.float32(127.0)
    u = jnp.bitwise_and(bits, 0x7FFFFF).astype(jnp.float32) * jnp.float32(2.0 ** -23)
    p = jnp.full((16,), _LOG2_COEF[0], jnp.float32)
    for c in _LOG2_COEF[1:]:
        p = p * u + jnp.float32(c)
    s = c2v * p + lv
    s = c2v * e_f + s
    return s


def _update(m, bi, idx, s):
    take = s > m
    bi = jnp.where(take, idx, bi)
    m = jnp.maximum(m, s)
    return m, bi


def _make_sampler():
    mesh = plsc.VectorSubcoreMesh(core_axis_name="c", subcore_axis_name="s")

    @functools.partial(
        pl.kernel,
        out_type=jax.ShapeDtypeStruct((B,), jnp.int32),
        mesh=mesh,
        scratch_types=[
            pltpu.VMEM((8, CK), jnp.float32),
            pltpu.VMEM((8, CK), jnp.float32),
            pltpu.VMEM((8, CK), jnp.float32),
            pltpu.VMEM((8, CK), jnp.float32),
            pltpu.VMEM((512,), jnp.float32),
            pltpu.VMEM((512,), jnp.float32),
            pltpu.VMEM((128,), jnp.float32),
            pltpu.VMEM((16,), jnp.float32),
            pltpu.VMEM((16,), jnp.int32),
            pltpu.VMEM((64,), jnp.float32),
            pltpu.VMEM((64,), jnp.int32),
            pltpu.VMEM_SHARED((256,), jnp.float32),
            pltpu.VMEM_SHARED((256,), jnp.int32),
            pltpu.SemaphoreType.DMA,
            pltpu.SemaphoreType.DMA,
            pltpu.SemaphoreType.DMA,
        ],
        compiler_params=pltpu.CompilerParams(needs_layout_passes=False),
    )
    def sampler(logits_hbm, temps_hbm, ltail_hbm, ntail_hbm, noise_hbm, out_hbm,
                lbuf0, lbuf1, nbuf0, nbuf1, ltail, ntail, tbuf,
                pmv, piv, mbuf, ibuf, spm_m, spm_i, sem0, sem1, sem2):
        cid = lax.axis_index("c")
        sid = lax.axis_index("s")
        grp = cid * 4 + sid // 4
        stripe = sid % 4
        row0 = grp * 8
        sbase = stripe * STRIPE

        lbufs = (lbuf0, lbuf1)
        nbufs = (nbuf0, nbuf1)
        sems = (sem0, sem1)
        lane = lax.iota(jnp.int32, 16)

        for r in range(8):
            pltpu.make_async_copy(
                temps_hbm.at[pl.ds((row0 + r) * 16, 16)],
                tbuf.at[pl.ds(r * 16, 16)], sem2).start()
        for r in range(8):
            pltpu.make_async_copy(
                temps_hbm.at[pl.ds(0, 16)],
                tbuf.at[pl.ds(r * 16, 16)], sem2).wait()
        c2vs = [tbuf[pl.ds(r * 16, 16)] * jnp.float32(NEG_LN2) for r in range(8)]

        pltpu.make_async_copy(
            ltail_hbm.at[pl.ds(grp * 512, 512)], ltail, sem2).start()
        pltpu.make_async_copy(
            ntail_hbm.at[pl.ds(grp * 512, 512)], ntail, sem2).start()
        pltpu.make_async_copy(
            ltail_hbm.at[pl.ds(0, 512)], ltail, sem2).wait()
        pltpu.make_async_copy(
            ntail_hbm.at[pl.ds(0, 512)], ntail, sem2).wait()

        def start_chunk(c, b):
            col = sbase + c * CK
            pltpu.make_async_copy(
                logits_hbm.at[pl.ds(row0, 8), pl.ds(col, CK)], lbufs[b], sems[b]).start()
            pltpu.make_async_copy(
                noise_hbm.at[pl.ds(row0, 8), pl.ds(col, CK)], nbufs[b], sems[b]).start()

        def wait_chunk(b):
            pltpu.make_async_copy(
                logits_hbm.at[pl.ds(0, 8), pl.ds(0, CK)], lbufs[b], sems[b]).wait()
            pltpu.make_async_copy(
                noise_hbm.at[pl.ds(0, 8), pl.ds(0, CK)], nbufs[b], sems[b]).wait()

        def scan_chunk(c, b, ms, cis):
            out_ms = []
            out_cis = []
            for r in range(8):
                @plsc.parallel_loop(0, CK, step=16 * UNROLL, unroll=1,
                                    carry=jnp.full((16,), -3.0e38, jnp.float32))
                def vbody(off, cm, _r=r, _b=b):
                    leaves = []
                    for k in range(UNROLL):
                        lv = lbufs[_b][_r, pl.ds(off + 16 * k, 16)]
                        nv = nbufs[_b][_r, pl.ds(off + 16 * k, 16)]
                        leaves.append(_score16(lv, nv, c2vs[_r]))
                    while len(leaves) > 1:
                        leaves = [jnp.maximum(leaves[a], leaves[a + 1])
                                  for a in range(0, len(leaves), 2)]
                    return jnp.maximum(cm, leaves[0])

                cm = vbody
                take = cm > ms[r]
                ci = jnp.where(take, jnp.full((16,), c, jnp.int32), cis[r])
                out_ms.append(jnp.maximum(ms[r], cm))
                out_cis.append(ci)
            return tuple(out_ms), tuple(out_cis)

        ms0 = tuple(jnp.full((16,), -3.0e38, jnp.float32) for _ in range(8))
        cis0 = tuple(jnp.zeros((16,), jnp.int32) for _ in range(8))

        start_chunk(0, 0)
        start_chunk(1, 1)

        def pair_body(cp, carry):
            ms, cis = carry
            c0 = cp * 2
            wait_chunk(0)
            ms, cis = scan_chunk(c0, 0, ms, cis)

            @pl.when(c0 + 2 < NCH)
            def _():
                start_chunk(c0 + 2, 0)

            wait_chunk(1)
            ms, cis = scan_chunk(c0 + 1, 1, ms, cis)

            @pl.when(c0 + 3 < NCH)
            def _():
                start_chunk(c0 + 3, 1)

            return ms, cis

        ms, cis = lax.fori_loop(0, NCH // 2, pair_body, (ms0, cis0))
        wait_chunk(0)
        ms, cis = scan_chunk(NCH - 1, 0, ms, cis)

        tmask = jnp.full((16,), stripe, jnp.int32) == jnp.int32(3)
        neg = jnp.full((16,), -3.0e38, jnp.float32)
        new_ms = []
        new_cis = []
        for r in range(8):
            cm = neg
            for j in range(4):
                lv = ltail[pl.ds(r * 64 + j * 16, 16)]
                nv = ntail[pl.ds(r * 64 + j * 16, 16)]
                s = _score16(lv, nv, c2vs[r])
                cm = jnp.maximum(cm, jnp.where(tmask, s, neg))
            take = cm > ms[r]
            new_cis.append(jnp.where(take, jnp.full((16,), NCH, jnp.int32), cis[r]))
            new_ms.append(jnp.maximum(ms[r], cm))
        ms, cis = tuple(new_ms), tuple(new_cis)

        pm = jnp.full((16,), -3.0e38, jnp.float32)
        pi = jnp.zeros((16,), jnp.int32)
        for r in range(8):
            gmax = jnp.max(ms[r])
            cbest = jnp.min(jnp.where(ms[r] == gmax, cis[r], jnp.int32(I32_MAX)))
            gv = jnp.full((16,), gmax, jnp.float32)
            col0 = sbase + jnp.minimum(cbest, jnp.int32(NCH - 1)) * CK
            pltpu.make_async_copy(
                logits_hbm.at[pl.ds(row0, 8), pl.ds(col0, CK)], lbufs[0], sems[0]).start()
            pltpu.make_async_copy(
                noise_hbm.at[pl.ds(row0, 8), pl.ds(col0, CK)], nbufs[0], sems[0]).start()
            wait_chunk(0)
            idx0 = lane + col0

            @plsc.parallel_loop(0, CK, step=16 * UNROLL, unroll=1,
                                carry=(jnp.full((16,), I32_MAX, jnp.int32), idx0))
            def rbody(off, car, _r=r):
                mn, ib = car
                for k in range(UNROLL):
                    lv = lbufs[0][_r, pl.ds(off + 16 * k, 16)]
                    nv = nbufs[0][_r, pl.ds(off + 16 * k, 16)]
                    s = _score16(lv, nv, c2vs[_r])
                    hit = s == gv
                    mn = jnp.minimum(mn, jnp.where(hit, ib + jnp.int32(16 * k), jnp.full((16,), I32_MAX, jnp.int32)))
                return mn, ib + jnp.int32(16 * UNROLL)

            mn, _ = rbody
            tmn = jnp.full((16,), I32_MAX, jnp.int32)
            for j in range(4):
                lv = ltail[pl.ds(r * 64 + j * 16, 16)]
                nv = ntail[pl.ds(r * 64 + j * 16, 16)]
                s = _score16(lv, nv, c2vs[r])
                hit = s == gv
                tmn = jnp.minimum(tmn, jnp.where(hit, lane + (MAIN_COLS + j * 16), jnp.full((16,), I32_MAX, jnp.int32)))
            mn = jnp.where(jnp.full((16,), cbest, jnp.int32) == NCH, tmn, mn)
            best = jnp.min(mn)
            pm = jnp.where(lane == r, jnp.full((16,), gmax, jnp.float32), pm)
            pi = jnp.where(lane == r, jnp.full((16,), best, jnp.int32), pi)

        pmv[...] = pm
        piv[...] = pi
        pltpu.sync_copy(pmv, spm_m.at[pl.ds(sid * 16, 16)])
        pltpu.sync_copy(piv, spm_i.at[pl.ds(sid * 16, 16)])
        plsc.subcore_barrier()

        @pl.when(stripe == 0)
        def _():
            gbase = (sid // 4) * 64
            pltpu.sync_copy(spm_m.at[pl.ds(gbase, 64)], mbuf)
            pltpu.sync_copy(spm_i.at[pl.ds(gbase, 64)], ibuf)
            m0 = mbuf[pl.ds(0, 16)]
            m1 = mbuf[pl.ds(16, 16)]
            m2 = mbuf[pl.ds(32, 16)]
            m3 = mbuf[pl.ds(48, 16)]
            i0 = ibuf[pl.ds(0, 16)]
            i1 = ibuf[pl.ds(16, 16)]
            i2 = ibuf[pl.ds(32, 16)]
            i3 = ibuf[pl.ds(48, 16)]

            def merge(ma, ia, mb, ib):
                takeb = jnp.logical_or(
                    mb > ma, jnp.logical_and(mb == ma, ib < ia))
                return jnp.where(takeb, mb, ma), jnp.where(takeb, ib, ia)

            ma, ia = merge(m0, i0, m1, i1)
            mb, ib = merge(m2, i2, m3, i3)
            _, fi = merge(ma, ia, mb, ib)
            piv[...] = fi
            pltpu.make_async_copy(
                piv.at[pl.ds(0, 8)], out_hbm.at[pl.ds(row0, 8)], sem2).start()
            pltpu.make_async_copy(
                piv.at[pl.ds(0, 8)], out_hbm.at[pl.ds(row0, 8)], sem2).wait()

    return sampler


_sampler = _make_sampler()


@jax.jit
def kernel(logits, temperatures, exp_noise):
    logits = logits.astype(jnp.float32)
    temps16 = jnp.broadcast_to(
        temperatures.astype(jnp.float32)[:, None], (B, 16)).reshape(B * 16)
    ltail = logits[:, MAIN_COLS:].reshape(B * TAIL_COLS)
    ntail = exp_noise[:, MAIN_COLS:].reshape(B * TAIL_COLS)
    return _sampler(logits, temps16, ltail, ntail, exp_noise)
